# baseline (speedup 1.0000x reference)
import functools

import jax
import jax.numpy as jnp
from jax import lax
from jax.experimental import pallas as pl
from jax.experimental.pallas import tpu as pltpu
from jax.experimental.pallas import tpu_sc as plsc

_VOCAB = 1000000
_T = 200
_EMB = 64
_B = 4096
_L = 200

_NTOK = _B * _L
_NW = 32
_TOK_PER_W = _NTOK // _NW
_PIECE = 128
_NP = 4
_CHUNK = _PIECE * _NP
_NCHUNK = _TOK_PER_W // _CHUNK
_BLK_PER_W = _TOK_PER_W // _PIECE
_NBLK = _NTOK // _PIECE


def _sc_body(idx_hbm, pos_hbm, msk_hbm, wtab_hbm, ptab_hbm, out_hbm,
             idx_s, pos_s, msk_s, buf, ptab, gsem):
    cid = lax.axis_index("c")
    sid = lax.axis_index("s")
    wid = sid * 2 + cid

    pltpu.sync_copy(ptab_hbm, ptab)

    iota = lax.iota(jnp.int32, 16)

    def chunk_body(g, carry):
        blk0 = wid * _BLK_PER_W + g * _NP
        pltpu.sync_copy(idx_hbm.at[pl.ds(blk0, _NP)], idx_s)
        pltpu.sync_copy(pos_hbm.at[pl.ds(blk0, _NP)], pos_s)
        pltpu.sync_copy(msk_hbm.at[pl.ds(blk0, _NP)], msk_s)
        for j in range(_NP):
            pltpu.async_copy(wtab_hbm.at[idx_s.at[j]],
                             buf.at[pl.ds(j * _PIECE, _PIECE)], gsem)
        for j in range(_NP):
            pltpu.make_async_copy(wtab_hbm.at[idx_s.at[j]],
                                  buf.at[pl.ds(j * _PIECE, _PIECE)],
                                  gsem).wait()

        def grp(gi, c2):
            j = gi // 8
            k = gi - j * 8
            idxv = idx_s[j, pl.ds(k * 16, 16)]
            posv = pos_s[j, pl.ds(k * 16, 16)]
            mv = msk_s[j, pl.ds(k * 16, 16)]
            nz = jnp.where(idxv == 2, 0.0, 1.0)
            sw = nz * mv
            rowv = iota + gi * 16
            pos64 = posv * 64
            for cc in range(_EMB):
                cv = jnp.full((16,), cc, jnp.int32)
                w = plsc.load_gather(buf, [rowv, cv])
                p = plsc.load_gather(ptab, [pos64 + cc])
                plsc.store_scatter(buf, [rowv, cv], w * sw + p * mv)
            return c2

        lax.fori_loop(0, _NP * 8, grp, 0)
        pltpu.sync_copy(buf, out_hbm.at[pl.ds(blk0 * _PIECE, _CHUNK)])
        return carry

    lax.fori_loop(0, _NCHUNK, chunk_body, 0)


_mesh = plsc.VectorSubcoreMesh(core_axis_name="c", subcore_axis_name="s")

_sc_call = pl.kernel(
    _sc_body,
    out_type=jax.ShapeDtypeStruct((_NTOK, _EMB), jnp.float32),
    mesh=_mesh,
    scratch_types=[
        pltpu.VMEM((_NP, _PIECE), jnp.int32),
        pltpu.VMEM((_NP, _PIECE), jnp.int32),
        pltpu.VMEM((_NP, _PIECE), jnp.float32),
        pltpu.VMEM((_CHUNK, _EMB), jnp.float32),
        pltpu.VMEM((_T * _EMB,), jnp.float32),
        pltpu.SemaphoreType.DMA,
    ],
    compiler_params=pltpu.CompilerParams(
        use_tc_tiling_on_sc=False, needs_layout_passes=False),
)


def kernel(inputs, mask, position, word_emb, position_emb):
    idx2d = inputs.reshape(_NBLK, _PIECE)
    pos2d = position.reshape(_NBLK, _PIECE)
    msk2d = mask.reshape(_NBLK, _PIECE)
    ptab_flat = position_emb.reshape(_T * _EMB)
    out = _sc_call(idx2d, pos2d, msk2d, word_emb, ptab_flat)
    return out.reshape(_B, _L, _EMB)


# speedup vs baseline: 1.0185x; 1.0185x over previous
import jax
import jax.numpy as jnp
from jax import lax
from jax.experimental import pallas as pl
from jax.experimental.pallas import tpu as pltpu
from jax.experimental.pallas import tpu_sc as plsc

_VOCAB = 1000000
_T = 200
_EMB = 64
_B = 4096
_L = 200

_NTOK = _B * _L
_NW = 32
_TOK_PER_W = _NTOK // _NW
_PIECE = 128
_NP = 2
_CHUNK = _PIECE * _NP
_NCHUNK = _TOK_PER_W // _CHUNK
_NPAIR = _NCHUNK // 2
_BLK_PER_W = _TOK_PER_W // _PIECE
_NBLK = _NTOK // _PIECE
_NGRP = _CHUNK // 16


def _sc_body(idx_hbm, pos_hbm, msk_hbm, wtab_hbm, ptab_hbm, out_hbm,
             idx_s, pos_s, msk_s, buf, bufo, ptab, gsem, osem):
    cid = lax.axis_index("c")
    sid = lax.axis_index("s")
    wid = sid * 2 + cid
    blk_base = wid * _BLK_PER_W

    pltpu.sync_copy(ptab_hbm, ptab)

    iota = lax.iota(jnp.int32, 16)

    def stage(g, par):
        blk0 = blk_base + g * _NP
        pltpu.sync_copy(idx_hbm.at[pl.ds(blk0, _NP)], idx_s[par])
        pltpu.sync_copy(pos_hbm.at[pl.ds(blk0, _NP)], pos_s[par])
        pltpu.sync_copy(msk_hbm.at[pl.ds(blk0, _NP)], msk_s[par])
        for j in range(_NP):
            pltpu.async_copy(wtab_hbm.at[idx_s[par].at[j]],
                             buf[par].at[pl.ds(j * _PIECE, _PIECE)],
                             gsem[par])

    def wait_gathers(par):
        for j in range(_NP):
            pltpu.make_async_copy(wtab_hbm.at[idx_s[par].at[j]],
                                  buf[par].at[pl.ds(j * _PIECE, _PIECE)],
                                  gsem[par]).wait()

    def compute(par):
        def grp(gi, c2):
            j = gi // 8
            k = gi - j * 8
            idxv = idx_s[par][j, pl.ds(k * 16, 16)]
            posv = pos_s[par][j, pl.ds(k * 16, 16)]
            mv = msk_s[par][j, pl.ds(k * 16, 16)]
            nz = jnp.where(idxv == 2, 0.0, 1.0)
            sw = nz * mv
            rowv = iota + gi * 16
            pos64 = posv * 64
            for cc in range(_EMB):
                cv = jnp.full((16,), cc, jnp.int32)
                w = plsc.load_gather(buf[par], [rowv, cv])
                p = plsc.load_gather(ptab, [pos64 + cc])
                plsc.store_scatter(bufo[par], [rowv, cv], w * sw + p * mv)
            return c2

        lax.fori_loop(0, _NGRP, grp, 0, unroll=2)

    def out_slice(g):
        return out_hbm.at[pl.ds((blk_base + g * _NP) * _PIECE, _CHUNK)]

    def fire_writeout(g, par):
        pltpu.async_copy(bufo[par], out_slice(g), osem[par])

    def wait_writeout(g, par):
        pltpu.make_async_copy(bufo[par], out_slice(g), osem[par]).wait()

    stage(0, 0)

    def pair_body(i, carry):
        g0 = i * 2
        g1 = g0 + 1
        stage(g1, 1)
        wait_gathers(0)

        @pl.when(i > 0)
        def _():
            wait_writeout(g0 - 2, 0)

        compute(0)
        fire_writeout(g0, 0)

        @pl.when(i < _NPAIR - 1)
        def _():
            stage(g0 + 2, 0)

        wait_gathers(1)

        @pl.when(i > 0)
        def _():
            wait_writeout(g1 - 2, 1)

        compute(1)
        fire_writeout(g1, 1)
        return carry

    lax.fori_loop(0, _NPAIR, pair_body, 0)
    wait_writeout(_NCHUNK - 2, 0)
    wait_writeout(_NCHUNK - 1, 1)


_mesh = plsc.VectorSubcoreMesh(core_axis_name="c", subcore_axis_name="s")

_sc_call = pl.kernel(
    _sc_body,
    out_type=jax.ShapeDtypeStruct((_NTOK, _EMB), jnp.float32),
    mesh=_mesh,
    scratch_types=[
        [pltpu.VMEM((_NP, _PIECE), jnp.int32) for _ in range(2)],
        [pltpu.VMEM((_NP, _PIECE), jnp.int32) for _ in range(2)],
        [pltpu.VMEM((_NP, _PIECE), jnp.float32) for _ in range(2)],
        [pltpu.VMEM((_CHUNK, _EMB), jnp.float32) for _ in range(2)],
        [pltpu.VMEM((_CHUNK, _EMB), jnp.float32) for _ in range(2)],
        pltpu.VMEM((_T * _EMB,), jnp.float32),
        [pltpu.SemaphoreType.DMA for _ in range(2)],
        [pltpu.SemaphoreType.DMA for _ in range(2)],
    ],
    compiler_params=pltpu.CompilerParams(
        use_tc_tiling_on_sc=False, needs_layout_passes=False),
)


def kernel(inputs, mask, position, word_emb, position_emb):
    idx2d = inputs.reshape(_NBLK, _PIECE)
    pos2d = position.reshape(_NBLK, _PIECE)
    msk2d = mask.reshape(_NBLK, _PIECE)
    ptab_flat = position_emb.reshape(_T * _EMB)
    out = _sc_call(idx2d, pos2d, msk2d, word_emb, ptab_flat)
    return out.reshape(_B, _L, _EMB)


# speedup vs baseline: 2.6855x; 2.6366x over previous
import jax
import jax.numpy as jnp
from jax import lax
from jax.experimental import pallas as pl
from jax.experimental.pallas import tpu as pltpu
from jax.experimental.pallas import tpu_sc as plsc

_VOCAB = 1000000
_T = 200
_EMB = 64
_B = 4096
_L = 200

_NTOK = _B * _L
_NW = 32
_TOK_PER_W = _NTOK // _NW
_PIECE = 128
_NP = 4
_CHUNK = _PIECE * _NP
_NCHUNK = _TOK_PER_W // _CHUNK
_NPAIR = _NCHUNK // 2
_BLK_PER_W = _TOK_PER_W // _PIECE
_NBLK = _NTOK // _PIECE
_NGRP = _CHUNK // 16


def _sc_body(idx_hbm, pos_hbm, msk_hbm, wtab_hbm, ptab_hbm, out_hbm,
             idx_s, pos_s, msk_s, buf, ptab, gsem, osem, ism):
    cid = lax.axis_index("c")
    sid = lax.axis_index("s")
    wid = sid * 2 + cid
    blk_base = wid * _BLK_PER_W

    pltpu.sync_copy(ptab_hbm, ptab)

    iota = lax.iota(jnp.int32, 16)
    coffs = [iota + q * 16 for q in range(4)]

    def idx_copies(g, par):
        blk0 = blk_base + g * _NP
        return [
            (idx_hbm.at[pl.ds(blk0, _NP)], idx_s[par]),
            (pos_hbm.at[pl.ds(blk0, _NP)], pos_s[par]),
            (msk_hbm.at[pl.ds(blk0, _NP)], msk_s[par]),
        ]

    def stage_idx(g, par):
        for src, dst in idx_copies(g, par):
            pltpu.async_copy(src, dst, ism[par])

    def wait_idx(g, par):
        for src, dst in idx_copies(g, par):
            pltpu.make_async_copy(src, dst, ism[par]).wait()

    def fire_gathers(g, par):
        for j in range(_NP):
            pltpu.async_copy(wtab_hbm.at[idx_s[par].at[j]],
                             buf[par].at[pl.ds(j * _PIECE, _PIECE)],
                             gsem[par])

    def wait_gathers(par):
        for j in range(_NP):
            pltpu.make_async_copy(wtab_hbm.at[idx_s[par].at[j]],
                                  buf[par].at[pl.ds(j * _PIECE, _PIECE)],
                                  gsem[par]).wait()

    def out_slice(g):
        return out_hbm.at[pl.ds((blk_base + g * _NP) * _PIECE, _CHUNK)]

    def fire_writeout(g, par):
        pltpu.async_copy(buf[par], out_slice(g), osem[par])

    def wait_writeout(g, par):
        pltpu.make_async_copy(buf[par], out_slice(g), osem[par]).wait()

    def splat(v, l):
        return jnp.take_along_axis(
            v, jnp.full((16,), l, jnp.int32), axis=0,
            mode="promise_in_bounds")

    def compute(par):
        def grp(gi, c2):
            j = gi // 8
            k = gi - j * 8
            idxv = idx_s[par][j, pl.ds(k * 16, 16)]
            posv = pos_s[par][j, pl.ds(k * 16, 16)]
            mv = msk_s[par][j, pl.ds(k * 16, 16)]
            nz = jnp.where(idxv == 2, 0.0, 1.0)
            sw = nz * mv
            pos64 = posv * 64
            all_one = (jnp.min(sw) == 1.0) & (jnp.max(sw) == 1.0)
            tokbase = gi * 16

            @pl.when(all_one)
            def _fast():
                for l in range(16):
                    psp = splat(pos64, l)
                    tok = tokbase + l
                    for q in range(4):
                        p = plsc.load_gather(ptab, [psp + coffs[q]])
                        plsc.addupdate(buf[par].at[tok, pl.ds(q * 16, 16)], p)

            @pl.when(jnp.logical_not(all_one))
            def _slow():
                for l in range(16):
                    psp = splat(pos64, l)
                    swsp = splat(sw, l)
                    msp = splat(mv, l)
                    tok = tokbase + l
                    for q in range(4):
                        w = buf[par][tok, pl.ds(q * 16, 16)]
                        p = plsc.load_gather(ptab, [psp + coffs[q]])
                        buf[par][tok, pl.ds(q * 16, 16)] = w * swsp + p * msp

            return c2

        lax.fori_loop(0, _NGRP, grp, 0)

    stage_idx(0, 0)
    wait_idx(0, 0)
    fire_gathers(0, 0)
    stage_idx(1, 1)

    def half(i, g, par):
        wait_gathers(par)
        compute(par)
        fire_writeout(g, par)

        @pl.when(g + 2 < _NCHUNK)
        def _():
            stage_idx(g + 2, par)

        @pl.when(g + 1 < _NCHUNK)
        def _():
            @pl.when(g > 0)
            def _():
                wait_writeout(g - 1, 1 - par)

            wait_idx(g + 1, 1 - par)
            fire_gathers(g + 1, 1 - par)

    def pair_body(i, carry):
        half(i, i * 2, 0)
        half(i, i * 2 + 1, 1)
        return carry

    lax.fori_loop(0, _NPAIR, pair_body, 0)
    wait_writeout(_NCHUNK - 2, 0)
    wait_writeout(_NCHUNK - 1, 1)


_mesh = plsc.VectorSubcoreMesh(core_axis_name="c", subcore_axis_name="s")

_sc_call = pl.kernel(
    _sc_body,
    out_type=jax.ShapeDtypeStruct((_NTOK, _EMB), jnp.float32),
    mesh=_mesh,
    scratch_types=[
        [pltpu.VMEM((_NP, _PIECE), jnp.int32) for _ in range(2)],
        [pltpu.VMEM((_NP, _PIECE), jnp.int32) for _ in range(2)],
        [pltpu.VMEM((_NP, _PIECE), jnp.float32) for _ in range(2)],
        [pltpu.VMEM((_CHUNK, _EMB), jnp.float32) for _ in range(2)],
        pltpu.VMEM((_T * _EMB,), jnp.float32),
        [pltpu.SemaphoreType.DMA for _ in range(2)],
        [pltpu.SemaphoreType.DMA for _ in range(2)],
        [pltpu.SemaphoreType.DMA for _ in range(2)],
    ],
    compiler_params=pltpu.CompilerParams(
        use_tc_tiling_on_sc=False, needs_layout_passes=False),
)


def kernel(inputs, mask, position, word_emb, position_emb):
    idx2d = inputs.reshape(_NBLK, _PIECE)
    pos2d = position.reshape(_NBLK, _PIECE)
    msk2d = mask.reshape(_NBLK, _PIECE)
    ptab_flat = position_emb.reshape(_T * _EMB)
    out = _sc_call(idx2d, pos2d, msk2d, word_emb, ptab_flat)
    return out.reshape(_B, _L, _EMB)


# speedup vs baseline: 3.1290x; 1.1652x over previous
import jax
import jax.numpy as jnp
from jax import lax
from jax.experimental import pallas as pl
from jax.experimental.pallas import tpu as pltpu
from jax.experimental.pallas import tpu_sc as plsc

_VOCAB = 1000000
_T = 200
_EMB = 64
_B = 4096
_L = 200

_NTOK = _B * _L
_NW = 32
_TOK_PER_W = _NTOK // _NW
_PIECE = 128
_NP = 4
_CHUNK = _PIECE * _NP
_NCHUNK = _TOK_PER_W // _CHUNK
_NPAIR = _NCHUNK // 2
_BLK_PER_W = _TOK_PER_W // _PIECE
_NBLK = _NTOK // _PIECE
_NGRP = _CHUNK // 16


def _sc_body(idx_hbm, pos_hbm, msk_hbm, wtab_hbm, ptab_hbm, out_hbm,
             idx_s, pos_s, msk_s, buf, ptab, gsem, osem, ism):
    cid = lax.axis_index("c")
    sid = lax.axis_index("s")
    wid = sid * 2 + cid
    blk_base = wid * _BLK_PER_W

    pltpu.sync_copy(ptab_hbm, ptab)

    iota = lax.iota(jnp.int32, 16)
    coffs = [iota + q * 16 for q in range(4)]

    def idx_copies(g, par):
        blk0 = blk_base + g * _NP
        return [
            (idx_hbm.at[pl.ds(blk0, _NP)], idx_s[par]),
            (pos_hbm.at[pl.ds(blk0, _NP)], pos_s[par]),
            (msk_hbm.at[pl.ds(blk0, _NP)], msk_s[par]),
        ]

    def stage_idx(g, par):
        for src, dst in idx_copies(g, par):
            pltpu.async_copy(src, dst, ism[par])

    def wait_idx(g, par):
        for src, dst in idx_copies(g, par):
            pltpu.make_async_copy(src, dst, ism[par]).wait()

    def fire_gathers(g, par):
        for j in range(_NP):
            pltpu.async_copy(wtab_hbm.at[idx_s[par].at[j]],
                             buf[par].at[pl.ds(j * _PIECE, _PIECE)],
                             gsem[par])

    def wait_gathers(par):
        for j in range(_NP):
            pltpu.make_async_copy(wtab_hbm.at[idx_s[par].at[j]],
                                  buf[par].at[pl.ds(j * _PIECE, _PIECE)],
                                  gsem[par]).wait()

    def out_slice(g):
        return out_hbm.at[pl.ds((blk_base + g * _NP) * _PIECE, _CHUNK)]

    def fire_writeout(g, par):
        pltpu.async_copy(buf[par], out_slice(g), osem[par])

    def wait_writeout(g, par):
        pltpu.make_async_copy(buf[par], out_slice(g), osem[par]).wait()

    def splat(v, l):
        return jnp.take_along_axis(
            v, jnp.full((16,), l, jnp.int32), axis=0,
            mode="promise_in_bounds")

    def compute(par):
        def grp(gi, c2):
            j = gi // 8
            k = gi - j * 8
            idxv = idx_s[par][j, pl.ds(k * 16, 16)]
            posv = pos_s[par][j, pl.ds(k * 16, 16)]
            mv = msk_s[par][j, pl.ds(k * 16, 16)]
            nz = jnp.where(idxv == 2, 0.0, 1.0)
            sw = nz * mv
            pos64 = posv * 64
            all_one = jnp.min(jnp.where(sw == 1.0, 1.0, 0.0)) == 1.0
            tokbase = gi * 16

            @pl.when(all_one)
            def _fast():
                psp = splat(pos64, 0)
                prev = [plsc.load_gather(ptab, [psp + coffs[q]])
                        for q in range(4)]
                for l in range(1, 16):
                    psp = splat(pos64, l)
                    cur = []
                    for q in range(4):
                        cur.append(plsc.load_gather(ptab, [psp + coffs[q]]))
                        plsc.addupdate(
                            buf[par].at[tokbase + l - 1, pl.ds(q * 16, 16)],
                            prev[q])
                    prev = cur
                for q in range(4):
                    plsc.addupdate(
                        buf[par].at[tokbase + 15, pl.ds(q * 16, 16)], prev[q])

            @pl.when(jnp.logical_not(all_one))
            def _slow():
                for l in range(16):
                    psp = splat(pos64, l)
                    swsp = splat(sw, l)
                    msp = splat(mv, l)
                    tok = tokbase + l
                    for q in range(4):
                        w = buf[par][tok, pl.ds(q * 16, 16)]
                        p = plsc.load_gather(ptab, [psp + coffs[q]])
                        buf[par][tok, pl.ds(q * 16, 16)] = w * swsp + p * msp

            return c2

        lax.fori_loop(0, _NGRP, grp, 0)

    stage_idx(0, 0)
    wait_idx(0, 0)
    fire_gathers(0, 0)
    stage_idx(1, 1)

    def half(i, g, par):
        wait_gathers(par)
        compute(par)
        fire_writeout(g, par)

        @pl.when(g + 2 < _NCHUNK)
        def _():
            stage_idx(g + 2, par)

        @pl.when(g + 1 < _NCHUNK)
        def _():
            @pl.when(g > 0)
            def _():
                wait_writeout(g - 1, 1 - par)

            wait_idx(g + 1, 1 - par)
            fire_gathers(g + 1, 1 - par)

    def pair_body(i, carry):
        half(i, i * 2, 0)
        half(i, i * 2 + 1, 1)
        return carry

    lax.fori_loop(0, _NPAIR, pair_body, 0)
    wait_writeout(_NCHUNK - 2, 0)
    wait_writeout(_NCHUNK - 1, 1)


_mesh = plsc.VectorSubcoreMesh(core_axis_name="c", subcore_axis_name="s")

_sc_call = pl.kernel(
    _sc_body,
    out_type=jax.ShapeDtypeStruct((_NTOK, _EMB), jnp.float32),
    mesh=_mesh,
    scratch_types=[
        [pltpu.VMEM((_NP, _PIECE), jnp.int32) for _ in range(2)],
        [pltpu.VMEM((_NP, _PIECE), jnp.int32) for _ in range(2)],
        [pltpu.VMEM((_NP, _PIECE), jnp.float32) for _ in range(2)],
        [pltpu.VMEM((_CHUNK, _EMB), jnp.float32) for _ in range(2)],
        pltpu.VMEM((_T * _EMB,), jnp.float32),
        [pltpu.SemaphoreType.DMA for _ in range(2)],
        [pltpu.SemaphoreType.DMA for _ in range(2)],
        [pltpu.SemaphoreType.DMA for _ in range(2)],
    ],
    compiler_params=pltpu.CompilerParams(
        use_tc_tiling_on_sc=False, needs_layout_passes=False),
)


def kernel(inputs, mask, position, word_emb, position_emb):
    idx2d = inputs.reshape(_NBLK, _PIECE)
    pos2d = position.reshape(_NBLK, _PIECE)
    msk2d = mask.reshape(_NBLK, _PIECE)
    ptab_flat = position_emb.reshape(_T * _EMB)
    out = _sc_call(idx2d, pos2d, msk2d, word_emb, ptab_flat)
    return out.reshape(_B, _L, _EMB)
